# Initial kernel scaffold; baseline (speedup 1.0000x reference)
import jax
import jax.numpy as jnp
from jax.experimental import pallas as pl


def kernel(x, edge_index, W1, b1, g1, be1, W2, b2, g2, be2, W3, b3, g3, be3, Wqi, bqi, Wqo, bqo, Wf, bf, gln, bln, Wc1, bc1, Wc2, bc2, Wc3, bc3):
    raise NotImplementedError("write your pallas kernel here")


# speedup vs baseline: 11.0089x; 11.0089x over previous
import functools
import math

import jax
import jax.numpy as jnp
from jax import lax
from jax.experimental import pallas as pl
from jax.experimental.pallas import tpu as pltpu
from jax.experimental.pallas import tpu_sc as plsc

N = 10000
D = 128
E = 320000
N_PAD = 10240
JUNK = N_PAD - N
NC, NS = 2, 16
NW = NC * NS
CHUNK = 128
K = -(-E // (NW * CHUNK))
E_PAD = NW * CHUNK * K
ROWS_PER_TILE = N_PAD // NS
BR = 512
NB = N_PAD // BR
IBN = 1.0 / math.sqrt(1.0 + 1e-5)
EPS = 1e-5


def _sc_count_body(dst_hbm, out_hbm, acc, dbuf, ones, zbuf):
    cid = lax.axis_index("c")
    tid = lax.axis_index("s")
    one = jnp.ones((16,), jnp.float32)
    z = jnp.zeros((16,), jnp.float32)
    for j in range(CHUNK // 16):
        ones[pl.ds(j * 16, 16)] = one

    def zb(i, carry):
        zbuf[pl.ds(i * 16, 16)] = z
        return carry

    lax.fori_loop(0, ROWS_PER_TILE // 16, zb, 0)
    pltpu.sync_copy(zbuf, acc.at[pl.ds(tid * ROWS_PER_TILE, ROWS_PER_TILE)])
    plsc.subcore_barrier()

    wbase = (cid * NS + tid) * K * CHUNK

    def body(k, carry):
        pltpu.sync_copy(dst_hbm.at[pl.ds(wbase + k * CHUNK, CHUNK)], dbuf)
        pltpu.sync_copy(ones, acc.at[dbuf], add=True)
        return carry

    lax.fori_loop(0, K, body, 0)
    plsc.subcore_barrier()
    pltpu.sync_copy(acc.at[pl.ds(tid * ROWS_PER_TILE, ROWS_PER_TILE)],
                    out_hbm.at[cid, pl.ds(tid * ROWS_PER_TILE, ROWS_PER_TILE)])


@functools.cache
def _sc_kernels():
    mesh = plsc.VectorSubcoreMesh(core_axis_name="c", subcore_axis_name="s",
                                  num_cores=NC, num_subcores=NS)
    sc_count = functools.partial(
        pl.kernel,
        out_type=jax.ShapeDtypeStruct((NC, N_PAD), jnp.float32),
        mesh=mesh,
        scratch_types=[
            pltpu.VMEM_SHARED((N_PAD,), jnp.float32),
            pltpu.VMEM((CHUNK,), jnp.int32),
            pltpu.VMEM((CHUNK,), jnp.float32),
            pltpu.VMEM((ROWS_PER_TILE,), jnp.float32),
        ],
    )(_sc_count_body)
    sc_scatter = functools.partial(
        pl.kernel,
        out_type=jax.ShapeDtypeStruct((NC, N_PAD, D), jnp.float32),
        mesh=mesh,
        scratch_types=[
            pltpu.VMEM_SHARED((N_PAD, D), jnp.float32),
            pltpu.VMEM((CHUNK,), jnp.int32),
            pltpu.VMEM((CHUNK,), jnp.int32),
            pltpu.VMEM((CHUNK, D), jnp.float32),
            pltpu.SemaphoreType.DMA,
        ],
    )(_sc_scatter_body)
    return sc_count, sc_scatter


def _sc_scatter_body(hs_hbm, src_hbm, dst_hbm, out_hbm, acc, sbuf, dbuf, rows, sem):
    cid = lax.axis_index("c")
    tid = lax.axis_index("s")
    z = jnp.zeros((16,), jnp.float32)

    def zrow(i, carry):
        for j in range(D // 16):
            rows[i, pl.ds(j * 16, 16)] = z
        return carry

    lax.fori_loop(0, CHUNK, zrow, 0)
    for j in range(ROWS_PER_TILE // CHUNK):
        pltpu.sync_copy(rows, acc.at[pl.ds(tid * ROWS_PER_TILE + j * CHUNK, CHUNK)])
    plsc.subcore_barrier()

    wbase = (cid * NS + tid) * K * CHUNK

    def body(k, carry):
        base = wbase + k * CHUNK
        pltpu.sync_copy(src_hbm.at[pl.ds(base, CHUNK)], sbuf)
        pltpu.sync_copy(dst_hbm.at[pl.ds(base, CHUNK)], dbuf)
        pltpu.async_copy(hs_hbm.at[sbuf], rows, sem).wait()
        pltpu.sync_copy(rows, acc.at[dbuf], add=True)
        return carry

    lax.fori_loop(0, K, body, 0)
    plsc.subcore_barrier()
    pltpu.sync_copy(acc.at[pl.ds(tid * ROWS_PER_TILE, ROWS_PER_TILE)],
                    out_hbm.at[cid, pl.ds(tid * ROWS_PER_TILE, ROWS_PER_TILE)])


def _tc_step1_body(x_ref, w_ref, cnt_ref, hs_ref):
    dinv = lax.rsqrt(cnt_ref[0] + cnt_ref[1] + 1.0)
    h = jnp.dot(x_ref[...], w_ref[...], preferred_element_type=jnp.float32)
    hs_ref[...] = h * dinv


_tc_step1 = pl.pallas_call(
    _tc_step1_body,
    grid=(NB,),
    in_specs=[
        pl.BlockSpec((BR, D), lambda i: (i, 0)),
        pl.BlockSpec((D, D), lambda i: (0, 0)),
        pl.BlockSpec((NC, BR, 1), lambda i: (0, i, 0)),
    ],
    out_specs=pl.BlockSpec((BR, D), lambda i: (i, 0)),
    out_shape=jax.ShapeDtypeStruct((N_PAD, D), jnp.float32),
)


def _tc_step_body(s_ref, hsp_ref, cnt_ref, b_ref, g_ref, be_ref, w_ref, hs_ref):
    dinv = lax.rsqrt(cnt_ref[0] + cnt_ref[1] + 1.0)
    agg = dinv * (s_ref[0] + s_ref[1] + hsp_ref[...])
    y = (agg + b_ref[...]) * IBN * g_ref[...] + be_ref[...]
    xn = jnp.maximum(y, 0.0)
    hs_ref[...] = jnp.dot(xn, w_ref[...], preferred_element_type=jnp.float32) * dinv


_tc_step = pl.pallas_call(
    _tc_step_body,
    grid=(NB,),
    in_specs=[
        pl.BlockSpec((NC, BR, D), lambda i: (0, i, 0)),
        pl.BlockSpec((BR, D), lambda i: (i, 0)),
        pl.BlockSpec((NC, BR, 1), lambda i: (0, i, 0)),
        pl.BlockSpec((1, D), lambda i: (0, 0)),
        pl.BlockSpec((1, D), lambda i: (0, 0)),
        pl.BlockSpec((1, D), lambda i: (0, 0)),
        pl.BlockSpec((D, D), lambda i: (0, 0)),
    ],
    out_specs=pl.BlockSpec((BR, D), lambda i: (i, 0)),
    out_shape=jax.ShapeDtypeStruct((N_PAD, D), jnp.float32),
)


def _tc_final_body(s_ref, hsp_ref, cnt_ref, b_ref, g_ref, be_ref,
                   wqi, bqi, wqo, bqo, wfg, wfq, bf, gln, bln,
                   wc1, bc1, wc2, bc2, wc3, bc3, out_ref, acc):
    i = pl.program_id(0)
    dinv = lax.rsqrt(cnt_ref[0] + cnt_ref[1] + 1.0)
    agg = dinv * (s_ref[0] + s_ref[1] + hsp_ref[...])
    y = (agg + b_ref[...]) * IBN * g_ref[...] + be_ref[...]
    rows = i * BR + lax.broadcasted_iota(jnp.int32, (BR, 1), 0)
    y = jnp.where(rows < N, y, 0.0)
    psum = jnp.sum(y, axis=0, keepdims=True)

    @pl.when(i == 0)
    def _init():
        acc[...] = psum

    @pl.when(i > 0)
    def _accum():
        acc[...] = acc[...] + psum

    @pl.when(i == NB - 1)
    def _head():
        graph = acc[...] * (1.0 / N)
        xe = jnp.tanh(jnp.dot(graph, wqi[...], preferred_element_type=jnp.float32) + bqi[...])
        q = jnp.tanh(xe)
        q = jnp.dot(q, wqo[...], preferred_element_type=jnp.float32) + bqo[...]
        fused = (jnp.dot(graph, wfg[...], preferred_element_type=jnp.float32)
                 + jnp.dot(q, wfq[...], preferred_element_type=jnp.float32) + bf[...])
        fused = jnp.maximum(fused, 0.0)
        mu = jnp.mean(fused, axis=-1, keepdims=True)
        var = jnp.mean((fused - mu) ** 2, axis=-1, keepdims=True)
        fused = (fused - mu) * lax.rsqrt(var + EPS) * gln[...] + bln[...]
        o = jnp.maximum(jnp.dot(fused, wc1[...], preferred_element_type=jnp.float32) + bc1[...], 0.0)
        o = jnp.maximum(jnp.dot(o, wc2[...], preferred_element_type=jnp.float32) + bc2[...], 0.0)
        out_ref[...] = jnp.dot(o, wc3[...], preferred_element_type=jnp.float32) + bc3[...]


def _full_spec(shape):
    nd = len(shape)
    return pl.BlockSpec(shape, lambda i, _nd=nd: (0,) * _nd)


_tc_final = pl.pallas_call(
    _tc_final_body,
    grid=(NB,),
    in_specs=[
        pl.BlockSpec((NC, BR, D), lambda i: (0, i, 0)),
        pl.BlockSpec((BR, D), lambda i: (i, 0)),
        pl.BlockSpec((NC, BR, 1), lambda i: (0, i, 0)),
        _full_spec((1, D)),
        _full_spec((1, D)),
        _full_spec((1, D)),
        _full_spec((D, 4)),
        _full_spec((1, 4)),
        _full_spec((4, 32)),
        _full_spec((1, 32)),
        _full_spec((D, D)),
        _full_spec((32, D)),
        _full_spec((1, D)),
        _full_spec((1, D)),
        _full_spec((1, D)),
        _full_spec((D, 64)),
        _full_spec((1, 64)),
        _full_spec((64, 32)),
        _full_spec((1, 32)),
        _full_spec((32, 1)),
        _full_spec((1, 1)),
    ],
    out_specs=pl.BlockSpec((1, 1), lambda i: (0, 0)),
    out_shape=jax.ShapeDtypeStruct((1, 1), jnp.float32),
    scratch_shapes=[pltpu.VMEM((1, D), jnp.float32)],
)


def kernel(x, edge_index, W1, b1, g1, be1, W2, b2, g2, be2, W3, b3, g3, be3,
           Wqi, bqi, Wqo, bqo, Wf, bf, gln, bln, Wc1, bc1, Wc2, bc2, Wc3, bc3):
    src = edge_index[0]
    dst = edge_index[1]
    pad = jnp.arange(E_PAD - E, dtype=jnp.int32)
    src_p = jnp.concatenate([src, pad % N])
    dst_p = jnp.concatenate([dst, N + pad % JUNK])
    x_p = jnp.pad(x, ((0, N_PAD - N), (0, 0)))

    _sc_count, _sc_scatter_rows = _sc_kernels()
    cnt = _sc_count(dst_p).reshape(NC, N_PAD, 1)

    hs1 = _tc_step1(x_p, W1, cnt)
    s1 = _sc_scatter_rows(hs1, src_p, dst_p)
    hs2 = _tc_step(s1, hs1, cnt, b1.reshape(1, D), g1.reshape(1, D),
                   be1.reshape(1, D), W2)
    s2 = _sc_scatter_rows(hs2, src_p, dst_p)
    hs3 = _tc_step(s2, hs2, cnt, b2.reshape(1, D), g2.reshape(1, D),
                   be2.reshape(1, D), W3)
    s3 = _sc_scatter_rows(hs3, src_p, dst_p)

    return _tc_final(
        s3, hs3, cnt, b3.reshape(1, D), g3.reshape(1, D), be3.reshape(1, D),
        Wqi, bqi.reshape(1, 4), Wqo, bqo.reshape(1, 32),
        Wf[:D], Wf[D:], bf.reshape(1, D), gln.reshape(1, D), bln.reshape(1, D),
        Wc1, bc1.reshape(1, 64), Wc2, bc2.reshape(1, 32), Wc3, bc3.reshape(1, 1))


# speedup vs baseline: 17.8028x; 1.6171x over previous
import functools
import math

import jax
import jax.numpy as jnp
from jax import lax
from jax.experimental import pallas as pl
from jax.experimental.pallas import tpu as pltpu
from jax.experimental.pallas import tpu_sc as plsc

N = 10000
D = 128
E = 320000
N_PAD = 10240
JUNK = N_PAD - N
NC, NS = 2, 16
NW = NC * NS
CHUNK = 128
K = 80
K2 = 160
NBUF = 4
DH = D // 2
E_PAD = NW * CHUNK * K
ROWS_PER_TILE = N_PAD // NS
BR = 512
NB = N_PAD // BR
IBN = 1.0 / math.sqrt(1.0 + 1e-5)
EPS = 1e-5


def _sc_count_body(dst_hbm, out_hbm, acc, dbuf, ones, zbuf, sem):
    cid = lax.axis_index("c")
    tid = lax.axis_index("s")
    one = jnp.ones((16,), jnp.float32)
    z = jnp.zeros((16,), jnp.float32)
    for j in range(CHUNK // 16):
        ones[pl.ds(j * 16, 16)] = one

    def zb(i, carry):
        zbuf[pl.ds(i * 16, 16)] = z
        return carry

    lax.fori_loop(0, ROWS_PER_TILE // 16, zb, 0)
    pltpu.sync_copy(zbuf, acc.at[pl.ds(tid * ROWS_PER_TILE, ROWS_PER_TILE)])
    pltpu.sync_copy(dst_hbm.at[tid, pl.ds(cid * K, K)], dbuf)
    plsc.subcore_barrier()

    def body(i, carry):
        for b in range(NBUF * 2):
            pltpu.async_copy(ones, acc.at[dbuf.at[i * NBUF * 2 + b]], sem,
                             add=True)
        for b in range(NBUF * 2):
            pltpu.make_async_copy(ones, acc.at[dbuf.at[i * NBUF * 2 + b]],
                                  sem).wait()
        return carry

    lax.fori_loop(0, K // (NBUF * 2), body, 0)
    plsc.subcore_barrier()
    pltpu.sync_copy(acc.at[pl.ds(tid * ROWS_PER_TILE, ROWS_PER_TILE)],
                    out_hbm.at[cid, pl.ds(tid * ROWS_PER_TILE, ROWS_PER_TILE)])


@functools.cache
def _sc_kernels():
    mesh = plsc.VectorSubcoreMesh(core_axis_name="c", subcore_axis_name="s",
                                  num_cores=NC, num_subcores=NS)
    sc_count = functools.partial(
        pl.kernel,
        out_type=jax.ShapeDtypeStruct((NC, N_PAD), jnp.float32),
        mesh=mesh,
        scratch_types=[
            pltpu.VMEM_SHARED((N_PAD,), jnp.float32),
            pltpu.VMEM((K, CHUNK), jnp.int32),
            pltpu.VMEM((CHUNK,), jnp.float32),
            pltpu.VMEM((ROWS_PER_TILE,), jnp.float32),
            pltpu.SemaphoreType.DMA,
        ],
    )(_sc_count_body)
    sc_scatter = functools.partial(
        pl.kernel,
        out_type=jax.ShapeDtypeStruct((NC, N_PAD, D), jnp.float32),
        mesh=mesh,
        scratch_types=(
            [pltpu.VMEM_SHARED((N_PAD, D), jnp.float32)]
            + [pltpu.VMEM((CHUNK,), jnp.int32)] * 8
            + [pltpu.VMEM((CHUNK, D), jnp.float32)] * 2
            + [pltpu.SemaphoreType.DMA] * 8
        ),
    )(_sc_scatter_body)
    return sc_count, sc_scatter


def _sc_scatter_body(hs_hbm, src_hbm, dst_hbm, out_hbm, acc,
                     si0, si1, si2, si3, di0, di1, di2, di3, r0, r1,
                     i0, i1, i2, i3, g0, g1, s0, s1):
    cid = lax.axis_index("c")
    tid = lax.axis_index("s")
    si = (si0, si1, si2, si3)
    di = (di0, di1, di2, di3)
    isems = (i0, i1, i2, i3)
    rbufs = (r0, r1)
    gsems = (g0, g1)
    ssems = (s0, s1)
    z = jnp.zeros((16,), jnp.float32)

    def zrow(i, carry):
        for j in range(D // 16):
            r0[i, pl.ds(j * 16, 16)] = z
        return carry

    lax.fori_loop(0, CHUNK, zrow, 0)
    for j in range(ROWS_PER_TILE // CHUNK):
        pltpu.sync_copy(r0, acc.at[pl.ds(tid * ROWS_PER_TILE + j * CHUNK, CHUNK)])

    def start_idx(k, sl):
        kg = cid * K + k
        pltpu.async_copy(src_hbm.at[tid, kg], si[sl], isems[sl])
        pltpu.async_copy(dst_hbm.at[tid, kg], di[sl], isems[sl])

    def wait_idx(k, sl):
        kg = cid * K + k
        pltpu.make_async_copy(src_hbm.at[tid, kg], si[sl], isems[sl]).wait()
        pltpu.make_async_copy(dst_hbm.at[tid, kg], di[sl], isems[sl]).wait()

    def start_g(sl, b):
        pltpu.async_copy(hs_hbm.at[si[sl]], rbufs[b], gsems[b])

    def wait_g(sl, b):
        pltpu.make_async_copy(hs_hbm.at[si[sl]], rbufs[b], gsems[b]).wait()

    def start_s(sl, b):
        pltpu.async_copy(rbufs[b], acc.at[di[sl]], ssems[b], add=True)

    def wait_s(sl, b):
        pltpu.make_async_copy(rbufs[b], acc.at[di[sl]], ssems[b]).wait()

    for sl in range(4):
        start_idx(sl, sl)
    plsc.subcore_barrier()
    wait_idx(0, 0)
    start_g(0, 0)
    wait_idx(1, 1)
    start_g(1, 1)

    def body(i, carry):
        c0 = i * 4
        wait_g(0, 0); start_s(0, 0)
        wait_g(1, 1); start_s(1, 1)
        wait_idx(c0 + 2, 2)
        wait_s(0, 0); start_g(2, 0); start_idx(c0 + 4, 0)
        wait_idx(c0 + 3, 3)
        wait_s(1, 1); start_g(3, 1); start_idx(c0 + 5, 1)
        wait_g(2, 0); start_s(2, 0)
        wait_g(3, 1); start_s(3, 1)
        wait_idx(c0 + 4, 0)
        wait_s(2, 0); start_g(0, 0); start_idx(c0 + 6, 2)
        wait_idx(c0 + 5, 1)
        wait_s(3, 1); start_g(1, 1); start_idx(c0 + 7, 3)
        return carry

    lax.fori_loop(0, K // 4 - 1, body, 0)
    wait_g(0, 0); start_s(0, 0)
    wait_g(1, 1); start_s(1, 1)
    wait_idx(K - 2, 2)
    wait_s(0, 0); start_g(2, 0)
    wait_idx(K - 1, 3)
    wait_s(1, 1); start_g(3, 1)
    wait_g(2, 0); start_s(2, 0)
    wait_g(3, 1); start_s(3, 1)
    wait_s(2, 0)
    wait_s(3, 1)
    plsc.subcore_barrier()
    pltpu.sync_copy(acc.at[pl.ds(tid * ROWS_PER_TILE, ROWS_PER_TILE)],
                    out_hbm.at[cid, pl.ds(tid * ROWS_PER_TILE, ROWS_PER_TILE)])


def _tc_step1_body(x_ref, w_ref, cnt_ref, hs_ref):
    dinv = lax.rsqrt(cnt_ref[0] + cnt_ref[1] + 1.0)
    h = jnp.dot(x_ref[...], w_ref[...], preferred_element_type=jnp.float32)
    hs_ref[...] = h * dinv


_tc_step1 = pl.pallas_call(
    _tc_step1_body,
    grid=(NB,),
    in_specs=[
        pl.BlockSpec((BR, D), lambda i: (i, 0)),
        pl.BlockSpec((D, D), lambda i: (0, 0)),
        pl.BlockSpec((NC, BR, 1), lambda i: (0, i, 0)),
    ],
    out_specs=pl.BlockSpec((BR, D), lambda i: (i, 0)),
    out_shape=jax.ShapeDtypeStruct((N_PAD, D), jnp.float32),
)


def _tc_step_body(s_ref, hsp_ref, cnt_ref, b_ref, g_ref, be_ref, w_ref, hs_ref):
    dinv = lax.rsqrt(cnt_ref[0] + cnt_ref[1] + 1.0)
    agg = dinv * (s_ref[0] + s_ref[1] + hsp_ref[...])
    y = (agg + b_ref[...]) * IBN * g_ref[...] + be_ref[...]
    xn = jnp.maximum(y, 0.0)
    hs_ref[...] = jnp.dot(xn, w_ref[...], preferred_element_type=jnp.float32) * dinv


_tc_step = pl.pallas_call(
    _tc_step_body,
    grid=(NB,),
    in_specs=[
        pl.BlockSpec((NC, BR, D), lambda i: (0, i, 0)),
        pl.BlockSpec((BR, D), lambda i: (i, 0)),
        pl.BlockSpec((NC, BR, 1), lambda i: (0, i, 0)),
        pl.BlockSpec((1, D), lambda i: (0, 0)),
        pl.BlockSpec((1, D), lambda i: (0, 0)),
        pl.BlockSpec((1, D), lambda i: (0, 0)),
        pl.BlockSpec((D, D), lambda i: (0, 0)),
    ],
    out_specs=pl.BlockSpec((BR, D), lambda i: (i, 0)),
    out_shape=jax.ShapeDtypeStruct((N_PAD, D), jnp.float32),
)


def _tc_final_body(s_ref, hsp_ref, cnt_ref, b_ref, g_ref, be_ref,
                   wqi, bqi, wqo, bqo, wfg, wfq, bf, gln, bln,
                   wc1, bc1, wc2, bc2, wc3, bc3, out_ref, acc):
    i = pl.program_id(0)
    dinv = lax.rsqrt(cnt_ref[0] + cnt_ref[1] + 1.0)
    agg = dinv * (s_ref[0] + s_ref[1] + hsp_ref[...])
    y = (agg + b_ref[...]) * IBN * g_ref[...] + be_ref[...]
    rows = i * BR + lax.broadcasted_iota(jnp.int32, (BR, 1), 0)
    y = jnp.where(rows < N, y, 0.0)
    psum = jnp.sum(y, axis=0, keepdims=True)

    @pl.when(i == 0)
    def _init():
        acc[...] = psum

    @pl.when(i > 0)
    def _accum():
        acc[...] = acc[...] + psum

    @pl.when(i == NB - 1)
    def _head():
        graph = acc[...] * (1.0 / N)
        xe = jnp.tanh(jnp.dot(graph, wqi[...], preferred_element_type=jnp.float32) + bqi[...])
        q = jnp.tanh(xe)
        q = jnp.dot(q, wqo[...], preferred_element_type=jnp.float32) + bqo[...]
        fused = (jnp.dot(graph, wfg[...], preferred_element_type=jnp.float32)
                 + jnp.dot(q, wfq[...], preferred_element_type=jnp.float32) + bf[...])
        fused = jnp.maximum(fused, 0.0)
        mu = jnp.mean(fused, axis=-1, keepdims=True)
        var = jnp.mean((fused - mu) ** 2, axis=-1, keepdims=True)
        fused = (fused - mu) * lax.rsqrt(var + EPS) * gln[...] + bln[...]
        o = jnp.maximum(jnp.dot(fused, wc1[...], preferred_element_type=jnp.float32) + bc1[...], 0.0)
        o = jnp.maximum(jnp.dot(o, wc2[...], preferred_element_type=jnp.float32) + bc2[...], 0.0)
        out_ref[...] = jnp.dot(o, wc3[...], preferred_element_type=jnp.float32) + bc3[...]


def _full_spec(shape):
    nd = len(shape)
    return pl.BlockSpec(shape, lambda i, _nd=nd: (0,) * _nd)


_tc_final = pl.pallas_call(
    _tc_final_body,
    grid=(NB,),
    in_specs=[
        pl.BlockSpec((NC, BR, D), lambda i: (0, i, 0)),
        pl.BlockSpec((BR, D), lambda i: (i, 0)),
        pl.BlockSpec((NC, BR, 1), lambda i: (0, i, 0)),
        _full_spec((1, D)),
        _full_spec((1, D)),
        _full_spec((1, D)),
        _full_spec((D, 4)),
        _full_spec((1, 4)),
        _full_spec((4, 32)),
        _full_spec((1, 32)),
        _full_spec((D, D)),
        _full_spec((32, D)),
        _full_spec((1, D)),
        _full_spec((1, D)),
        _full_spec((1, D)),
        _full_spec((D, 64)),
        _full_spec((1, 64)),
        _full_spec((64, 32)),
        _full_spec((1, 32)),
        _full_spec((32, 1)),
        _full_spec((1, 1)),
    ],
    out_specs=pl.BlockSpec((1, 1), lambda i: (0, 0)),
    out_shape=jax.ShapeDtypeStruct((1, 1), jnp.float32),
    scratch_shapes=[pltpu.VMEM((1, D), jnp.float32)],
)


def kernel(x, edge_index, W1, b1, g1, be1, W2, b2, g2, be2, W3, b3, g3, be3,
           Wqi, bqi, Wqo, bqo, Wf, bf, gln, bln, Wc1, bc1, Wc2, bc2, Wc3, bc3):
    src = edge_index[0]
    dst = edge_index[1]
    pad = jnp.arange(E_PAD - E, dtype=jnp.int32)
    src_p = jnp.concatenate([src, pad % N])
    dst_p = jnp.concatenate([dst, N + pad % JUNK])
    src_t = src_p.reshape(NS, K2, CHUNK)
    dst_t = dst_p.reshape(NS, K2, CHUNK)
    x_p = jnp.pad(x, ((0, N_PAD - N), (0, 0)))

    _sc_count, _sc_scatter_rows = _sc_kernels()
    cnt = _sc_count(dst_t).reshape(NC, N_PAD, 1)

    hs1 = _tc_step1(x_p, W1, cnt)
    s1 = _sc_scatter_rows(hs1, src_t, dst_t)
    hs2 = _tc_step(s1, hs1, cnt, b1.reshape(1, D), g1.reshape(1, D),
                   be1.reshape(1, D), W2)
    s2 = _sc_scatter_rows(hs2, src_t, dst_t)
    hs3 = _tc_step(s2, hs2, cnt, b2.reshape(1, D), g2.reshape(1, D),
                   be2.reshape(1, D), W3)
    s3 = _sc_scatter_rows(hs3, src_t, dst_t)

    return _tc_final(
        s3, hs3, cnt, b3.reshape(1, D), g3.reshape(1, D), be3.reshape(1, D),
        Wqi, bqi.reshape(1, 4), Wqo, bqo.reshape(1, 32),
        Wf[:D], Wf[D:], bf.reshape(1, D), gln.reshape(1, D), bln.reshape(1, D),
        Wc1, bc1.reshape(1, 64), Wc2, bc2.reshape(1, 32), Wc3, bc3.reshape(1, 1))


# speedup vs baseline: 24.6599x; 1.3852x over previous
import functools
import math

import jax
import jax.numpy as jnp
from jax import lax
from jax.experimental import pallas as pl
from jax.experimental.pallas import tpu as pltpu
from jax.experimental.pallas import tpu_sc as plsc

N = 10000
D = 128
E = 320000
N_PAD = 10240
JUNK = N_PAD - N
NC, NS = 2, 16
NW = NC * NS
CHUNK = 128
K = 80
K2 = 160
NBUF = 4
DH = D // 2
E_PAD = NW * CHUNK * K
ROWS_PER_TILE = N_PAD // NS
BR = 512
NB = N_PAD // BR
IBN = 1.0 / math.sqrt(1.0 + 1e-5)
EPS = 1e-5


def _sc_count_body(dst_hbm, out_hbm, acc, dbuf, ones, zbuf, sem):
    cid = lax.axis_index("c")
    tid = lax.axis_index("s")
    one = jnp.ones((16,), jnp.float32)
    z = jnp.zeros((16,), jnp.float32)
    for j in range(CHUNK // 16):
        ones[pl.ds(j * 16, 16)] = one

    def zb(i, carry):
        zbuf[pl.ds(i * 16, 16)] = z
        return carry

    lax.fori_loop(0, ROWS_PER_TILE // 16, zb, 0)
    pltpu.sync_copy(zbuf, acc.at[pl.ds(tid * ROWS_PER_TILE, ROWS_PER_TILE)])
    pltpu.sync_copy(dst_hbm.at[tid, pl.ds(cid * K, K)], dbuf)
    plsc.subcore_barrier()

    def body(i, carry):
        for b in range(NBUF * 2):
            pltpu.async_copy(ones, acc.at[dbuf.at[i * NBUF * 2 + b]], sem,
                             add=True)
        for b in range(NBUF * 2):
            pltpu.make_async_copy(ones, acc.at[dbuf.at[i * NBUF * 2 + b]],
                                  sem).wait()
        return carry

    lax.fori_loop(0, K // (NBUF * 2), body, 0)
    plsc.subcore_barrier()
    pltpu.sync_copy(acc.at[pl.ds(tid * ROWS_PER_TILE, ROWS_PER_TILE)],
                    out_hbm.at[cid, pl.ds(tid * ROWS_PER_TILE, ROWS_PER_TILE)])


@functools.cache
def _sc_kernels():
    mesh = plsc.VectorSubcoreMesh(core_axis_name="c", subcore_axis_name="s",
                                  num_cores=NC, num_subcores=NS)
    sc_count = functools.partial(
        pl.kernel,
        out_type=jax.ShapeDtypeStruct((NC, N_PAD), jnp.float32),
        mesh=mesh,
        scratch_types=[
            pltpu.VMEM_SHARED((N_PAD,), jnp.float32),
            pltpu.VMEM((K, CHUNK), jnp.int32),
            pltpu.VMEM((CHUNK,), jnp.float32),
            pltpu.VMEM((ROWS_PER_TILE,), jnp.float32),
            pltpu.SemaphoreType.DMA,
        ],
    )(_sc_count_body)
    sc_scatter = functools.partial(
        pl.kernel,
        out_type=jax.ShapeDtypeStruct((NC, N_PAD, D), jnp.float32),
        mesh=mesh,
        scratch_types=(
            [pltpu.VMEM_SHARED((N_ACC, D), jnp.float32)]
            + [pltpu.VMEM((CHUNK,), jnp.int32)] * 8
            + [pltpu.VMEM((CHUNK, D), jnp.float32)] * 3
            + [pltpu.SemaphoreType.DMA] * 10
        ),
    )(_sc_scatter_body)
    return sc_count, sc_scatter


N_ACC = 10112
RPT_ACC = N_ACC // NS


def _sc_scatter_body(hs_hbm, src_hbm, dst_hbm, out_hbm, acc,
                     si0, si1, si2, si3, di0, di1, di2, di3, r0, r1, r2,
                     i0, i1, i2, i3, g0, g1, g2, s0, s1, s2):
    cid = lax.axis_index("c")
    tid = lax.axis_index("s")
    si = (si0, si1, si2, si3)
    di = (di0, di1, di2, di3)
    isems = (i0, i1, i2, i3)
    rbufs = (r0, r1, r2)
    gsems = (g0, g1, g2)
    ssems = (s0, s1, s2)
    z = jnp.zeros((16,), jnp.float32)

    def start_idx(k, q):
        kg = cid * K + k
        pltpu.async_copy(src_hbm.at[tid, kg], si[q], isems[q])
        pltpu.async_copy(dst_hbm.at[tid, kg], di[q], isems[q])

    def wait_idx(k, q):
        kg = cid * K + k
        pltpu.make_async_copy(src_hbm.at[tid, kg], si[q], isems[q]).wait()
        pltpu.make_async_copy(dst_hbm.at[tid, kg], di[q], isems[q]).wait()

    def start_g(q, b):
        pltpu.async_copy(hs_hbm.at[si[q]], rbufs[b], gsems[b])

    def wait_g(q, b):
        pltpu.make_async_copy(hs_hbm.at[si[q]], rbufs[b], gsems[b]).wait()

    def start_s(q, b):
        pltpu.async_copy(rbufs[b], acc.at[di[q]], ssems[b], add=True)

    def wait_s(q, b):
        pltpu.make_async_copy(rbufs[b], acc.at[di[q]], ssems[b]).wait()

    for q in range(4):
        start_idx(q, q)

    def zrow(i, carry):
        for j in range(D // 16):
            r0[i, pl.ds(j * 16, 16)] = z
        return carry

    lax.fori_loop(0, CHUNK, zrow, 0)
    for j in range(4):
        pltpu.sync_copy(r0, acc.at[pl.ds(tid * RPT_ACC + j * CHUNK, CHUNK)])
    pltpu.sync_copy(r0.at[pl.ds(0, RPT_ACC - 4 * CHUNK)],
                    acc.at[pl.ds(tid * RPT_ACC + 4 * CHUNK, RPT_ACC - 4 * CHUNK)])
    plsc.subcore_barrier()

    def steady(k, with_prefetch=True):
        b, q = k % 3, k % 4
        wait_idx(k, q)
        wait_s((k - 3) % 4, (k - 3) % 3)
        if with_prefetch:
            start_idx(k + 1, (k + 1) % 4)
        start_g(q, b)
        wait_g((k - 2) % 4, (k - 2) % 3)
        start_s((k - 2) % 4, (k - 2) % 3)

    wait_idx(0, 0); start_g(0, 0)
    wait_idx(1, 1); start_g(1, 1)
    wait_idx(2, 2); start_g(2, 2); wait_g(0, 0); start_s(0, 0)
    for k in range(3, 12):
        steady(k)

    def body(i, carry):
        for j in range(12):
            steady_static[j](i * 12 + 12 + j)
        return carry

    steady_static = []
    for j in range(12):
        b, q = (12 + j) % 3, (12 + j) % 4

        def mk(b=b, q=q, j=j):
            def f(k):
                wait_idx(k, q)
                wait_s((q - 3) % 4, (b - 3) % 3)
                start_idx(k + 1, (q + 1) % 4)
                start_g(q, b)
                wait_g((q - 2) % 4, (b - 2) % 3)
                start_s((q - 2) % 4, (b - 2) % 3)
            return f
        steady_static.append(mk())

    lax.fori_loop(0, (K - 20) // 12, body, 0)
    for k in range(K - 8, K - 1):
        steady(k)
    steady(K - 1, with_prefetch=False)
    wait_g((K - 2) % 4, (K - 2) % 3); start_s((K - 2) % 4, (K - 2) % 3)
    wait_g((K - 1) % 4, (K - 1) % 3); start_s((K - 1) % 4, (K - 1) % 3)
    wait_s((K - 3) % 4, (K - 3) % 3)
    wait_s((K - 2) % 4, (K - 2) % 3)
    wait_s((K - 1) % 4, (K - 1) % 3)
    plsc.subcore_barrier()
    pltpu.sync_copy(acc.at[pl.ds(tid * RPT_ACC, RPT_ACC)],
                    out_hbm.at[cid, pl.ds(tid * RPT_ACC, RPT_ACC)])


def _tc_step1_body(x_ref, w_ref, cnt_ref, hs_ref):
    dinv = lax.rsqrt(cnt_ref[0] + cnt_ref[1] + 1.0)
    h = jnp.dot(x_ref[...], w_ref[...], preferred_element_type=jnp.float32)
    hs_ref[...] = h * dinv


_tc_step1 = pl.pallas_call(
    _tc_step1_body,
    grid=(NB,),
    in_specs=[
        pl.BlockSpec((BR, D), lambda i: (i, 0)),
        pl.BlockSpec((D, D), lambda i: (0, 0)),
        pl.BlockSpec((NC, BR, 1), lambda i: (0, i, 0)),
    ],
    out_specs=pl.BlockSpec((BR, D), lambda i: (i, 0)),
    out_shape=jax.ShapeDtypeStruct((N_PAD, D), jnp.float32),
)


def _tc_step_body(s_ref, hsp_ref, cnt_ref, b_ref, g_ref, be_ref, w_ref, hs_ref):
    dinv = lax.rsqrt(cnt_ref[0] + cnt_ref[1] + 1.0)
    agg = dinv * (s_ref[0] + s_ref[1] + hsp_ref[...])
    y = (agg + b_ref[...]) * IBN * g_ref[...] + be_ref[...]
    xn = jnp.maximum(y, 0.0)
    hs_ref[...] = jnp.dot(xn, w_ref[...], preferred_element_type=jnp.float32) * dinv


_tc_step = pl.pallas_call(
    _tc_step_body,
    grid=(NB,),
    in_specs=[
        pl.BlockSpec((NC, BR, D), lambda i: (0, i, 0)),
        pl.BlockSpec((BR, D), lambda i: (i, 0)),
        pl.BlockSpec((NC, BR, 1), lambda i: (0, i, 0)),
        pl.BlockSpec((1, D), lambda i: (0, 0)),
        pl.BlockSpec((1, D), lambda i: (0, 0)),
        pl.BlockSpec((1, D), lambda i: (0, 0)),
        pl.BlockSpec((D, D), lambda i: (0, 0)),
    ],
    out_specs=pl.BlockSpec((BR, D), lambda i: (i, 0)),
    out_shape=jax.ShapeDtypeStruct((N_PAD, D), jnp.float32),
)


def _tc_final_body(s_ref, hsp_ref, cnt_ref, b_ref, g_ref, be_ref,
                   wqi, bqi, wqo, bqo, wfg, wfq, bf, gln, bln,
                   wc1, bc1, wc2, bc2, wc3, bc3, out_ref, acc):
    i = pl.program_id(0)
    dinv = lax.rsqrt(cnt_ref[0] + cnt_ref[1] + 1.0)
    agg = dinv * (s_ref[0] + s_ref[1] + hsp_ref[...])
    y = (agg + b_ref[...]) * IBN * g_ref[...] + be_ref[...]
    rows = i * BR + lax.broadcasted_iota(jnp.int32, (BR, 1), 0)
    y = jnp.where(rows < N, y, 0.0)
    psum = jnp.sum(y, axis=0, keepdims=True)

    @pl.when(i == 0)
    def _init():
        acc[...] = psum

    @pl.when(i > 0)
    def _accum():
        acc[...] = acc[...] + psum

    @pl.when(i == NB - 1)
    def _head():
        graph = acc[...] * (1.0 / N)
        xe = jnp.tanh(jnp.dot(graph, wqi[...], preferred_element_type=jnp.float32) + bqi[...])
        q = jnp.tanh(xe)
        q = jnp.dot(q, wqo[...], preferred_element_type=jnp.float32) + bqo[...]
        fused = (jnp.dot(graph, wfg[...], preferred_element_type=jnp.float32)
                 + jnp.dot(q, wfq[...], preferred_element_type=jnp.float32) + bf[...])
        fused = jnp.maximum(fused, 0.0)
        mu = jnp.mean(fused, axis=-1, keepdims=True)
        var = jnp.mean((fused - mu) ** 2, axis=-1, keepdims=True)
        fused = (fused - mu) * lax.rsqrt(var + EPS) * gln[...] + bln[...]
        o = jnp.maximum(jnp.dot(fused, wc1[...], preferred_element_type=jnp.float32) + bc1[...], 0.0)
        o = jnp.maximum(jnp.dot(o, wc2[...], preferred_element_type=jnp.float32) + bc2[...], 0.0)
        out_ref[...] = jnp.dot(o, wc3[...], preferred_element_type=jnp.float32) + bc3[...]


def _full_spec(shape):
    nd = len(shape)
    return pl.BlockSpec(shape, lambda i, _nd=nd: (0,) * _nd)


_tc_final = pl.pallas_call(
    _tc_final_body,
    grid=(NB,),
    in_specs=[
        pl.BlockSpec((NC, BR, D), lambda i: (0, i, 0)),
        pl.BlockSpec((BR, D), lambda i: (i, 0)),
        pl.BlockSpec((NC, BR, 1), lambda i: (0, i, 0)),
        _full_spec((1, D)),
        _full_spec((1, D)),
        _full_spec((1, D)),
        _full_spec((D, 4)),
        _full_spec((1, 4)),
        _full_spec((4, 32)),
        _full_spec((1, 32)),
        _full_spec((D, D)),
        _full_spec((32, D)),
        _full_spec((1, D)),
        _full_spec((1, D)),
        _full_spec((1, D)),
        _full_spec((D, 64)),
        _full_spec((1, 64)),
        _full_spec((64, 32)),
        _full_spec((1, 32)),
        _full_spec((32, 1)),
        _full_spec((1, 1)),
    ],
    out_specs=pl.BlockSpec((1, 1), lambda i: (0, 0)),
    out_shape=jax.ShapeDtypeStruct((1, 1), jnp.float32),
    scratch_shapes=[pltpu.VMEM((1, D), jnp.float32)],
)


def kernel(x, edge_index, W1, b1, g1, be1, W2, b2, g2, be2, W3, b3, g3, be3,
           Wqi, bqi, Wqo, bqo, Wf, bf, gln, bln, Wc1, bc1, Wc2, bc2, Wc3, bc3):
    src = edge_index[0]
    dst = edge_index[1]
    pad = jnp.arange(E_PAD - E, dtype=jnp.int32)
    src_p = jnp.concatenate([src, pad % N])
    dst_p = jnp.concatenate([dst, N + pad % (N_ACC - N)])
    src_t = src_p.reshape(NS, K2, CHUNK)
    dst_t = dst_p.reshape(NS, K2, CHUNK)
    x_p = jnp.pad(x, ((0, N_PAD - N), (0, 0)))

    _sc_count, _sc_scatter_rows = _sc_kernels()
    cnt = _sc_count(dst_t).reshape(NC, N_PAD, 1)

    hs1 = _tc_step1(x_p, W1, cnt)
    s1 = _sc_scatter_rows(hs1, src_t, dst_t)
    hs2 = _tc_step(s1, hs1, cnt, b1.reshape(1, D), g1.reshape(1, D),
                   be1.reshape(1, D), W2)
    s2 = _sc_scatter_rows(hs2, src_t, dst_t)
    hs3 = _tc_step(s2, hs2, cnt, b2.reshape(1, D), g2.reshape(1, D),
                   be2.reshape(1, D), W3)
    s3 = _sc_scatter_rows(hs3, src_t, dst_t)

    return _tc_final(
        s3, hs3, cnt, b3.reshape(1, D), g3.reshape(1, D), be3.reshape(1, D),
        Wqi, bqi.reshape(1, 4), Wqo, bqo.reshape(1, 32),
        Wf[:D], Wf[D:], bf.reshape(1, D), gln.reshape(1, D), bln.reshape(1, D),
        Wc1, bc1.reshape(1, 64), Wc2, bc2.reshape(1, 32), Wc3, bc3.reshape(1, 1))


# speedup vs baseline: 26.1259x; 1.0594x over previous
import functools
import math

import jax
import jax.numpy as jnp
from jax import lax
from jax.experimental import pallas as pl
from jax.experimental.pallas import tpu as pltpu
from jax.experimental.pallas import tpu_sc as plsc

N = 10000
D = 128
E = 320000
N_PAD = 10240
JUNK = N_PAD - N
NC, NS = 2, 16
NW = NC * NS
CHUNK = 128
K = 80
K2 = 160
NBUF = 4
DH = D // 2
E_PAD = NW * CHUNK * K
ROWS_PER_TILE = N_PAD // NS
BR = 512
NB = N_PAD // BR
IBN = 1.0 / math.sqrt(1.0 + 1e-5)
EPS = 1e-5


def _sc_count_body(ei_hbm, out_hbm, acc, dbuf, ones, zbuf, sem):
    cid = lax.axis_index("c")
    tid = lax.axis_index("s")
    one = jnp.ones((16,), jnp.float32)
    z = jnp.zeros((16,), jnp.float32)
    for j in range(CHUNK // 16):
        ones[pl.ds(j * 16, 16)] = one

    def zb(i, carry):
        zbuf[pl.ds(i * 16, 16)] = z
        return carry

    lax.fori_loop(0, ROWS_PER_TILE // 16, zb, 0)
    pltpu.sync_copy(zbuf, acc.at[pl.ds(tid * ROWS_PER_TILE, ROWS_PER_TILE)])
    pltpu.sync_copy(ei_hbm.at[1, tid, pl.ds(cid * K, K)], dbuf)
    plsc.subcore_barrier()

    def body(i, carry):
        for b in range(NBUF * 2):
            pltpu.async_copy(ones, acc.at[dbuf.at[i * NBUF * 2 + b]], sem,
                             add=True)
        for b in range(NBUF * 2):
            pltpu.make_async_copy(ones, acc.at[dbuf.at[i * NBUF * 2 + b]],
                                  sem).wait()
        return carry

    lax.fori_loop(0, K // (NBUF * 2), body, 0)
    plsc.subcore_barrier()
    pltpu.sync_copy(acc.at[pl.ds(tid * ROWS_PER_TILE, ROWS_PER_TILE)],
                    out_hbm.at[cid, pl.ds(tid * ROWS_PER_TILE, ROWS_PER_TILE)])


@functools.cache
def _sc_kernels():
    mesh = plsc.VectorSubcoreMesh(core_axis_name="c", subcore_axis_name="s",
                                  num_cores=NC, num_subcores=NS)
    sc_count = functools.partial(
        pl.kernel,
        out_type=jax.ShapeDtypeStruct((NC, N_PAD), jnp.float32),
        mesh=mesh,
        scratch_types=[
            pltpu.VMEM_SHARED((N_PAD,), jnp.float32),
            pltpu.VMEM((K, CHUNK), jnp.int32),
            pltpu.VMEM((CHUNK,), jnp.float32),
            pltpu.VMEM((ROWS_PER_TILE,), jnp.float32),
            pltpu.SemaphoreType.DMA,
        ],
    )(_sc_count_body)
    sc_scatter = functools.partial(
        pl.kernel,
        out_type=jax.ShapeDtypeStruct((NC, N_PAD, D), jnp.float32),
        mesh=mesh,
        scratch_types=(
            [pltpu.VMEM_SHARED((N_ACC, D), jnp.float32)]
            + [pltpu.VMEM((CHUNK,), jnp.int32)] * 8
            + [pltpu.VMEM((CHUNK, D), jnp.float32)] * 3
            + [pltpu.SemaphoreType.DMA] * 10
        ),
    )(_sc_scatter_body)
    return sc_count, sc_scatter


N_ACC = 10112
RPT_ACC = N_ACC // NS


def _sc_scatter_body(hs_hbm, ei_hbm, out_hbm, acc,
                     si0, si1, si2, si3, di0, di1, di2, di3, r0, r1, r2,
                     i0, i1, i2, i3, g0, g1, g2, s0, s1, s2):
    cid = lax.axis_index("c")
    tid = lax.axis_index("s")
    si = (si0, si1, si2, si3)
    di = (di0, di1, di2, di3)
    isems = (i0, i1, i2, i3)
    rbufs = (r0, r1, r2)
    gsems = (g0, g1, g2)
    ssems = (s0, s1, s2)
    z = jnp.zeros((16,), jnp.float32)

    def start_idx(k, q):
        kg = cid * K + k
        pltpu.async_copy(ei_hbm.at[0, tid, kg], si[q], isems[q])
        pltpu.async_copy(ei_hbm.at[1, tid, kg], di[q], isems[q])

    def wait_idx(k, q):
        kg = cid * K + k
        pltpu.make_async_copy(ei_hbm.at[0, tid, kg], si[q], isems[q]).wait()
        pltpu.make_async_copy(ei_hbm.at[1, tid, kg], di[q], isems[q]).wait()

    def start_g(q, b):
        pltpu.async_copy(hs_hbm.at[si[q]], rbufs[b], gsems[b])

    def wait_g(q, b):
        pltpu.make_async_copy(hs_hbm.at[si[q]], rbufs[b], gsems[b]).wait()

    def start_s(q, b):
        pltpu.async_copy(rbufs[b], acc.at[di[q]], ssems[b], add=True)

    def wait_s(q, b):
        pltpu.make_async_copy(rbufs[b], acc.at[di[q]], ssems[b]).wait()

    for q in range(4):
        start_idx(q, q)

    def zrow(i, carry):
        for j in range(D // 16):
            r0[i, pl.ds(j * 16, 16)] = z
        return carry

    lax.fori_loop(0, CHUNK, zrow, 0)
    for j in range(4):
        pltpu.sync_copy(r0, acc.at[pl.ds(tid * RPT_ACC + j * CHUNK, CHUNK)])
    pltpu.sync_copy(r0.at[pl.ds(0, RPT_ACC - 4 * CHUNK)],
                    acc.at[pl.ds(tid * RPT_ACC + 4 * CHUNK, RPT_ACC - 4 * CHUNK)])
    plsc.subcore_barrier()

    def steady(k, with_prefetch=True):
        b, q = k % 3, k % 4
        wait_idx(k, q)
        wait_s((k - 3) % 4, (k - 3) % 3)
        if with_prefetch:
            start_idx(k + 1, (k + 1) % 4)
        start_g(q, b)
        wait_g((k - 2) % 4, (k - 2) % 3)
        start_s((k - 2) % 4, (k - 2) % 3)

    wait_idx(0, 0); start_g(0, 0)
    wait_idx(1, 1); start_g(1, 1)
    wait_idx(2, 2); start_g(2, 2); wait_g(0, 0); start_s(0, 0)
    for k in range(3, 12):
        steady(k)

    def body(i, carry):
        for j in range(12):
            steady_static[j](i * 12 + 12 + j)
        return carry

    steady_static = []
    for j in range(12):
        b, q = (12 + j) % 3, (12 + j) % 4

        def mk(b=b, q=q, j=j):
            def f(k):
                wait_idx(k, q)
                wait_s((q - 3) % 4, (b - 3) % 3)
                start_idx(k + 1, (q + 1) % 4)
                start_g(q, b)
                wait_g((q - 2) % 4, (b - 2) % 3)
                start_s((q - 2) % 4, (b - 2) % 3)
            return f
        steady_static.append(mk())

    lax.fori_loop(0, (K - 20) // 12, body, 0)
    for k in range(K - 8, K - 1):
        steady(k)
    steady(K - 1, with_prefetch=False)
    wait_g((K - 2) % 4, (K - 2) % 3); start_s((K - 2) % 4, (K - 2) % 3)
    wait_g((K - 1) % 4, (K - 1) % 3); start_s((K - 1) % 4, (K - 1) % 3)
    wait_s((K - 3) % 4, (K - 3) % 3)
    wait_s((K - 2) % 4, (K - 2) % 3)
    wait_s((K - 1) % 4, (K - 1) % 3)
    plsc.subcore_barrier()
    pltpu.sync_copy(acc.at[pl.ds(tid * RPT_ACC, RPT_ACC)],
                    out_hbm.at[cid, pl.ds(tid * RPT_ACC, RPT_ACC)])


def _dinv_col(cnt_ref):
    dinv = lax.rsqrt(cnt_ref[0] + cnt_ref[1] + 1.0)
    return lax.broadcast_in_dim(dinv, (BR, 1), (0,))


def _tc_step1_body(x_ref, w_ref, cnt_ref, hs_ref):
    dinv = _dinv_col(cnt_ref)
    h = jnp.dot(x_ref[...], w_ref[...], preferred_element_type=jnp.float32)
    hs_ref[...] = h * dinv


_tc_step1 = pl.pallas_call(
    _tc_step1_body,
    grid=(NB,),
    in_specs=[
        pl.BlockSpec((BR, D), lambda i: (i, 0)),
        pl.BlockSpec((D, D), lambda i: (0, 0)),
        pl.BlockSpec((NC, BR), lambda i: (0, i)),
    ],
    out_specs=pl.BlockSpec((BR, D), lambda i: (i, 0)),
    out_shape=jax.ShapeDtypeStruct((N_PAD, D), jnp.float32),
)


def _tc_step_body(s_ref, hsp_ref, cnt_ref, b_ref, g_ref, be_ref, w_ref, hs_ref):
    dinv = _dinv_col(cnt_ref)
    agg = dinv * (s_ref[0] + s_ref[1] + hsp_ref[...])
    y = (agg + b_ref[...]) * IBN * g_ref[...] + be_ref[...]
    xn = jnp.maximum(y, 0.0)
    hs_ref[...] = jnp.dot(xn, w_ref[...], preferred_element_type=jnp.float32) * dinv


_tc_step = pl.pallas_call(
    _tc_step_body,
    grid=(NB,),
    in_specs=[
        pl.BlockSpec((NC, BR, D), lambda i: (0, i, 0)),
        pl.BlockSpec((BR, D), lambda i: (i, 0)),
        pl.BlockSpec((NC, BR), lambda i: (0, i)),
        pl.BlockSpec((1, D), lambda i: (0, 0)),
        pl.BlockSpec((1, D), lambda i: (0, 0)),
        pl.BlockSpec((1, D), lambda i: (0, 0)),
        pl.BlockSpec((D, D), lambda i: (0, 0)),
    ],
    out_specs=pl.BlockSpec((BR, D), lambda i: (i, 0)),
    out_shape=jax.ShapeDtypeStruct((N_PAD, D), jnp.float32),
)


def _tc_final_body(s_ref, hsp_ref, cnt_ref, b_ref, g_ref, be_ref,
                   wqi, bqi, wqo, bqo, wfg, wfq, bf, gln, bln,
                   wc1, bc1, wc2, bc2, wc3, bc3, out_ref, acc):
    i = pl.program_id(0)
    dinv = _dinv_col(cnt_ref)
    agg = dinv * (s_ref[0] + s_ref[1] + hsp_ref[...])
    y = (agg + b_ref[...]) * IBN * g_ref[...] + be_ref[...]
    rows = i * BR + lax.broadcasted_iota(jnp.int32, (BR, 1), 0)
    y = jnp.where(rows < N, y, 0.0)
    psum = jnp.sum(y, axis=0, keepdims=True)

    @pl.when(i == 0)
    def _init():
        acc[...] = psum

    @pl.when(i > 0)
    def _accum():
        acc[...] = acc[...] + psum

    @pl.when(i == NB - 1)
    def _head():
        graph = acc[...] * (1.0 / N)
        xe = jnp.tanh(jnp.dot(graph, wqi[...], preferred_element_type=jnp.float32) + bqi[...])
        q = jnp.tanh(xe)
        q = jnp.dot(q, wqo[...], preferred_element_type=jnp.float32) + bqo[...]
        fused = (jnp.dot(graph, wfg[...], preferred_element_type=jnp.float32)
                 + jnp.dot(q, wfq[...], preferred_element_type=jnp.float32) + bf[...])
        fused = jnp.maximum(fused, 0.0)
        mu = jnp.mean(fused, axis=-1, keepdims=True)
        var = jnp.mean((fused - mu) ** 2, axis=-1, keepdims=True)
        fused = (fused - mu) * lax.rsqrt(var + EPS) * gln[...] + bln[...]
        o = jnp.maximum(jnp.dot(fused, wc1[...], preferred_element_type=jnp.float32) + bc1[...], 0.0)
        o = jnp.maximum(jnp.dot(o, wc2[...], preferred_element_type=jnp.float32) + bc2[...], 0.0)
        out_ref[...] = jnp.dot(o, wc3[...], preferred_element_type=jnp.float32) + bc3[...]


def _full_spec(shape):
    nd = len(shape)
    return pl.BlockSpec(shape, lambda i, _nd=nd: (0,) * _nd)


_tc_final = pl.pallas_call(
    _tc_final_body,
    grid=(NB,),
    in_specs=[
        pl.BlockSpec((NC, BR, D), lambda i: (0, i, 0)),
        pl.BlockSpec((BR, D), lambda i: (i, 0)),
        pl.BlockSpec((NC, BR), lambda i: (0, i)),
        _full_spec((1, D)),
        _full_spec((1, D)),
        _full_spec((1, D)),
        _full_spec((D, 4)),
        _full_spec((1, 4)),
        _full_spec((4, 32)),
        _full_spec((1, 32)),
        _full_spec((D, D)),
        _full_spec((32, D)),
        _full_spec((1, D)),
        _full_spec((1, D)),
        _full_spec((1, D)),
        _full_spec((D, 64)),
        _full_spec((1, 64)),
        _full_spec((64, 32)),
        _full_spec((1, 32)),
        _full_spec((32, 1)),
        _full_spec((1, 1)),
    ],
    out_specs=pl.BlockSpec((1, 1), lambda i: (0, 0)),
    out_shape=jax.ShapeDtypeStruct((1, 1), jnp.float32),
    scratch_shapes=[pltpu.VMEM((1, D), jnp.float32)],
)


def kernel(x, edge_index, W1, b1, g1, be1, W2, b2, g2, be2, W3, b3, g3, be3,
           Wqi, bqi, Wqo, bqo, Wf, bf, gln, bln, Wc1, bc1, Wc2, bc2, Wc3, bc3):
    pad = jnp.arange(E_PAD - E, dtype=jnp.int32)
    pads = jnp.stack([pad % N, N + pad % (N_ACC - N)])
    ei_t = jnp.concatenate([edge_index, pads], axis=1).reshape(2, NS, K2, CHUNK)
    x_p = jnp.pad(x, ((0, N_PAD - N), (0, 0)))

    _sc_count, _sc_scatter_rows = _sc_kernels()
    cnt = _sc_count(ei_t)

    hs1 = _tc_step1(x_p, W1, cnt)
    s1 = _sc_scatter_rows(hs1, ei_t)
    hs2 = _tc_step(s1, hs1, cnt, b1.reshape(1, D), g1.reshape(1, D),
                   be1.reshape(1, D), W2)
    s2 = _sc_scatter_rows(hs2, ei_t)
    hs3 = _tc_step(s2, hs2, cnt, b2.reshape(1, D), g2.reshape(1, D),
                   be2.reshape(1, D), W3)
    s3 = _sc_scatter_rows(hs3, ei_t)

    return _tc_final(
        s3, hs3, cnt, b3.reshape(1, D), g3.reshape(1, D), be3.reshape(1, D),
        Wqi, bqi.reshape(1, 4), Wqo, bqo.reshape(1, 32),
        Wf[:D], Wf[D:], bf.reshape(1, D), gln.reshape(1, D), bln.reshape(1, D),
        Wc1, bc1.reshape(1, 64), Wc2, bc2.reshape(1, 32), Wc3, bc3.reshape(1, 1))
